# Initial kernel scaffold; baseline (speedup 1.0000x reference)
import jax
import jax.numpy as jnp
from jax.experimental import pallas as pl


def kernel(x, edge_index, edge_attr, W_edge, W, b):
    raise NotImplementedError("write your pallas kernel here")


# speedup vs baseline: 3.0656x; 3.0656x over previous
import functools

import jax
import jax.numpy as jnp
from jax import lax
from jax.experimental import pallas as pl
from jax.experimental.pallas import tpu as pltpu
from jax.experimental.pallas import tpu_sc as plsc

_NUM_LAYERS = 3
_D = 128
_N = 10000
_E = 320000

_NC = 2
_NS = 16
_NW = _NC * _NS
_CHUNK = 128
_N_CHUNKS = _E // _CHUNK
_ITERS = -(-_N_CHUNKS // _NW)
_N_PAD = 10240
_ROWS_PER_SUB = _N_PAD // _NS


def _proj_body(ea_ref, w_ref, o_ref):
    o_ref[...] = jnp.dot(ea_ref[...], w_ref[...],
                         preferred_element_type=jnp.float32)


def _edge_project(edge_attr, w_edge_l):
    blk = 8000
    return pl.pallas_call(
        _proj_body,
        grid=(_E // blk,),
        in_specs=[
            pl.BlockSpec((blk, 16), lambda i: (i, 0)),
            pl.BlockSpec((16, _D), lambda i: (0, 0)),
        ],
        out_specs=pl.BlockSpec((blk, _D), lambda i: (i, 0)),
        out_shape=jax.ShapeDtypeStruct((_E, _D), jnp.float32),
    )(edge_attr, w_edge_l)


def _update_body(x_ref, a0_ref, a1_ref, w_ref, b_ref, o_ref):
    h = x_ref[...] + a0_ref[...] + a1_ref[...]
    y = jnp.dot(h, w_ref[...], preferred_element_type=jnp.float32) + b_ref[...]
    o_ref[...] = jnp.maximum(y, 0.0)


def _node_update(x, agg0, agg1, w_l, b_l):
    blk = 2000
    return pl.pallas_call(
        _update_body,
        grid=(_N // blk,),
        in_specs=[
            pl.BlockSpec((blk, _D), lambda i: (i, 0)),
            pl.BlockSpec((blk, _D), lambda i: (i, 0)),
            pl.BlockSpec((blk, _D), lambda i: (i, 0)),
            pl.BlockSpec((_D, _D), lambda i: (0, 0)),
            pl.BlockSpec((1, _D), lambda i: (0, 0)),
        ],
        out_specs=pl.BlockSpec((blk, _D), lambda i: (i, 0)),
        out_shape=jax.ShapeDtypeStruct((_N, _D), jnp.float32),
    )(x, agg0, agg1, w_l, b_l.reshape(1, _D))


def _sc_edge_body(x_hbm, e_hbm, src_hbm, dst_hbm, out_hbm,
                  src_v, dst_v, rows_v, e_v, agg_sh, sem):
    cid = lax.axis_index("c")
    sid = lax.axis_index("s")
    wid = cid * _NS + sid

    @pl.loop(0, _CHUNK)
    def _zero_tile(r):
        for j in range(0, _D, 16):
            rows_v[r, pl.ds(j, 16)] = jnp.zeros((16,), jnp.float32)

    @pl.loop(0, _ROWS_PER_SUB, step=_CHUNK)
    def _zero_agg(r):
        pltpu.sync_copy(rows_v, agg_sh.at[pl.ds(sid * _ROWS_PER_SUB + r, _CHUNK)])

    plsc.subcore_barrier()

    @pl.loop(0, _ITERS)
    def _edge_chunk(i):
        ci = wid + _NW * i

        @pl.when(ci < _N_CHUNKS)
        def _():
            base = ci * _CHUNK
            pltpu.sync_copy(src_hbm.at[pl.ds(base, _CHUNK)], src_v)
            pltpu.sync_copy(dst_hbm.at[pl.ds(base, _CHUNK)], dst_v)
            pltpu.async_copy(x_hbm.at[src_v], rows_v, sem).wait()
            pltpu.sync_copy(e_hbm.at[pl.ds(base, _CHUNK)], e_v)

            @pl.loop(0, _CHUNK)
            def _combine(r):
                for j in range(0, _D, 16):
                    v = rows_v[r, pl.ds(j, 16)] + e_v[r, pl.ds(j, 16)]
                    rows_v[r, pl.ds(j, 16)] = jnp.maximum(v, 0.0)

            pltpu.sync_copy(rows_v, agg_sh.at[dst_v], add=True)

    plsc.subcore_barrier()

    @pl.loop(0, _ROWS_PER_SUB, step=_CHUNK)
    def _copy_out(r):
        off = sid * _ROWS_PER_SUB + r
        pltpu.sync_copy(agg_sh.at[pl.ds(off, _CHUNK)],
                        out_hbm.at[cid, pl.ds(off, _CHUNK)])


@functools.cache
def _sc_edge_pass():
    return pl.kernel(
        _sc_edge_body,
        out_type=jax.ShapeDtypeStruct((_NC, _N_PAD, _D), jnp.float32),
        mesh=plsc.VectorSubcoreMesh(core_axis_name="c", subcore_axis_name="s",
                                    num_cores=_NC, num_subcores=_NS),
        scratch_types=[
            pltpu.VMEM((_CHUNK,), jnp.int32),
            pltpu.VMEM((_CHUNK,), jnp.int32),
            pltpu.VMEM((_CHUNK, _D), jnp.float32),
            pltpu.VMEM((_CHUNK, _D), jnp.float32),
            pltpu.VMEM_SHARED((_N_PAD, _D), jnp.float32),
            pltpu.SemaphoreType.DMA,
        ],
    )


def kernel(x, edge_index, edge_attr, W_edge, W, b):
    src = edge_index[0].astype(jnp.int32)
    dst = edge_index[1].astype(jnp.int32)
    x = x.astype(jnp.float32)
    for l in range(_NUM_LAYERS):
        e = _edge_project(edge_attr, W_edge[l])
        agg = _sc_edge_pass()(x, e, src, dst)
        x = _node_update(x, agg[0, :_N], agg[1, :_N], W[l], b[l])
    return x
